# baseline (speedup 1.0000x reference)
import jax
import jax.numpy as jnp
from jax import lax
from jax.experimental import pallas as pl
from jax.experimental.pallas import tpu as pltpu
from jax.experimental.pallas import tpu_sc as plsc

B = 16384
D = 32
V = 1000000
LANES = 16
NUM_WORKERS = 32
B_PER_W = B // NUM_WORKERS
IDX_CHUNK = 128
N_CHUNKS = B_PER_W // IDX_CHUNK
HALF = B_PER_W // 2
N_CH_HALF = N_CHUNKS // 2


def _sc_body(t_idx_hbm, c_idx_hbm, t_tab_hbm, c_tab_hbm, out_hbm,
             t_idx_v, c_idx_v, t_q_v, c_q_v, t_rows_v, c_rows_v, out_v, sem):
    nc = 2
    wid = lax.axis_index("s") * nc + lax.axis_index("c")
    blk = wid * N_CHUNKS

    pltpu.sync_copy(t_idx_hbm.at[pl.ds(blk, N_CHUNKS)], t_idx_v)
    pltpu.sync_copy(c_idx_hbm.at[pl.ds(blk, N_CHUNKS)], c_idx_v)

    def mkq(i, _):
        for j in range(N_CHUNKS):
            tv = t_idx_v[j, pl.ds(i * LANES, LANES)]
            cv = c_idx_v[j, pl.ds(i * LANES, LANES)]
            t_q_v[j, pl.ds(i * LANES, LANES)] = lax.shift_right_logical(
                tv, jnp.int32(2))
            c_q_v[j, pl.ds(i * LANES, LANES)] = lax.shift_right_logical(
                cv, jnp.int32(2))
        return 0

    lax.fori_loop(0, IDX_CHUNK // LANES, mkq, 0)

    lanes = lax.iota(jnp.int32, LANES)

    for half in range(2):
        cps = []
        for j in range(N_CH_HALF):
            jj = half * N_CH_HALF + j
            cps.append(pltpu.async_copy(
                t_tab_hbm.at[t_q_v.at[jj]],
                t_rows_v.at[pl.ds(j * IDX_CHUNK, IDX_CHUNK)], sem))
            cps.append(pltpu.async_copy(
                c_tab_hbm.at[c_q_v.at[jj]],
                c_rows_v.at[pl.ds(j * IDX_CHUNK, IDX_CHUNK)], sem))
        for cp in cps:
            cp.wait()

        def group(g, _):
            rows = jnp.int32(LANES) * g + lanes
            k = half * (HALF // LANES) + g
            tvcol = t_idx_v[k // 8, pl.ds((k % 8) * LANES, LANES)]
            cvcol = c_idx_v[k // 8, pl.ds((k % 8) * LANES, LANES)]
            t_base = (tvcol & jnp.int32(3)) * jnp.int32(D)
            c_base = (cvcol & jnp.int32(3)) * jnp.int32(D)
            accs = [jnp.zeros((LANES,), jnp.float32) for _ in range(4)]
            for d in range(D):
                tv = plsc.load_gather(t_rows_v, [rows, t_base + jnp.int32(d)])
                cv = plsc.load_gather(c_rows_v, [rows, c_base + jnp.int32(d)])
                accs[d % 4] = accs[d % 4] + tv * cv
            out_v[pl.ds(half * HALF + g * LANES, LANES)] = (
                (accs[0] + accs[1]) + (accs[2] + accs[3]))
            return 0

        lax.fori_loop(0, HALF // LANES, group, 0)

    pltpu.sync_copy(out_v, out_hbm.at[pl.ds(wid * B_PER_W, B_PER_W)])


@jax.jit
def _run(t_idx, c_idx, t_tab, c_tab):
    mesh = plsc.VectorSubcoreMesh(core_axis_name="c", subcore_axis_name="s")
    return pl.kernel(
        _sc_body,
        out_type=jax.ShapeDtypeStruct((B,), jnp.float32),
        mesh=mesh,
        compiler_params=pltpu.CompilerParams(needs_layout_passes=False),
        scratch_types=[
            pltpu.VMEM((N_CHUNKS, IDX_CHUNK), jnp.int32),
            pltpu.VMEM((N_CHUNKS, IDX_CHUNK), jnp.int32),
            pltpu.VMEM((N_CHUNKS, IDX_CHUNK), jnp.int32),
            pltpu.VMEM((N_CHUNKS, IDX_CHUNK), jnp.int32),
            pltpu.VMEM((HALF, 128), jnp.float32),
            pltpu.VMEM((HALF, 128), jnp.float32),
            pltpu.VMEM((B_PER_W,), jnp.float32),
            pltpu.SemaphoreType.DMA,
        ],
    )(t_idx, c_idx, t_tab, c_tab)


def kernel(t_kmer, c_kmer, label, T_weight, C_weight):
    del label
    t_idx = t_kmer.astype(jnp.int32).reshape(B // IDX_CHUNK, IDX_CHUNK)
    c_idx = c_kmer.astype(jnp.int32).reshape(B // IDX_CHUNK, IDX_CHUNK)
    return _run(t_idx, c_idx,
                T_weight.reshape(V // 4, 128), C_weight.reshape(V // 4, 128))


# speedup vs baseline: 3.3460x; 3.3460x over previous
import jax
import jax.numpy as jnp
from jax import lax
from jax.experimental import pallas as pl
from jax.experimental.pallas import tpu as pltpu
from jax.experimental.pallas import tpu_sc as plsc

B = 16384
D = 32
V = 1000000
LANES = 16
NUM_WORKERS = 32
B_PER_W = B // NUM_WORKERS
IDX_CHUNK = 128
N_CHUNKS = B_PER_W // IDX_CHUNK
PANEL = 128
TAIL_START = (V // PANEL) * PANEL
TAIL_W = V - TAIL_START
LAST_PANEL = TAIL_START - PANEL
CHUNK8 = 8


def _sc_body(t_idx_hbm, c_idx_hbm, t_tab_hbm, c_tab_hbm, out_hbm,
             t_idx_v, c_idx_v, panel_buf, tcols_lo, tcols_hi, pbuf,
             t_tail, c_tail, out_v, sem):
    nc = 2
    wid = lax.axis_index("s") * nc + lax.axis_index("c")
    blk = wid * N_CHUNKS

    pltpu.sync_copy(t_idx_hbm.at[pl.ds(blk, N_CHUNKS)], t_idx_v)
    pltpu.sync_copy(c_idx_hbm.at[pl.ds(blk, N_CHUNKS)], c_idx_v)
    pltpu.sync_copy(t_tab_hbm.at[:, pl.ds(TAIL_START, TAIL_W)], t_tail)
    pltpu.sync_copy(c_tab_hbm.at[:, pl.ds(TAIL_START, TAIL_W)], c_tail)

    d_lo = lax.iota(jnp.int32, LANES)
    d_hi = d_lo + jnp.int32(LANES)

    def column(tail_ref, buf_slot, vs):
        col = jnp.minimum(vs & jnp.int32(PANEL - 1), jnp.int32(PANEL - 1))
        tcol = jnp.minimum(
            jnp.maximum(vs - jnp.int32(TAIL_START), jnp.int32(0)),
            jnp.int32(TAIL_W - 1))
        is_tail = jnp.broadcast_to(vs >= jnp.int32(TAIL_START), (LANES,))
        colv = jnp.broadcast_to(col, (LANES,))
        tcolv = jnp.broadcast_to(tcol, (LANES,))
        lo = jnp.where(
            is_tail,
            plsc.load_gather(tail_ref, [d_lo, tcolv]),
            plsc.load_gather(buf_slot, [d_lo, colv]))
        hi = jnp.where(
            is_tail,
            plsc.load_gather(tail_ref, [d_hi, tcolv]),
            plsc.load_gather(buf_slot, [d_hi, colv]))
        return lo, hi

    def panel_start(vs):
        return jnp.minimum(
            lax.shift_right_logical(vs, jnp.int32(7)) * jnp.int32(PANEL),
            jnp.int32(LAST_PANEL))

    def round16(g, _):
        tvec = t_idx_v[g // 8, pl.ds((g % 8) * LANES, LANES)]
        cvec = c_idx_v[g // 8, pl.ds((g % 8) * LANES, LANES)]
        t_s = [lax.squeeze(lax.slice(tvec, (u,), (u + 1,)), (0,))
               for u in range(LANES)]
        c_s = [lax.squeeze(lax.slice(cvec, (u,), (u + 1,)), (0,))
               for u in range(LANES)]

        cps = [pltpu.async_copy(
            t_tab_hbm.at[:, pl.ds(pl.multiple_of(panel_start(t_s[u]), PANEL),
                                  PANEL)],
            panel_buf.at[u], sem) for u in range(LANES)]
        for cp in cps:
            cp.wait()
        for u in range(LANES):
            lo, hi = column(t_tail, panel_buf.at[u], t_s[u])
            tcols_lo[u] = lo
            tcols_hi[u] = hi

        cps = [pltpu.async_copy(
            c_tab_hbm.at[:, pl.ds(pl.multiple_of(panel_start(c_s[u]), PANEL),
                                  PANEL)],
            panel_buf.at[u], sem) for u in range(LANES)]
        for cp in cps:
            cp.wait()
        for u in range(LANES):
            c_lo, c_hi = column(c_tail, panel_buf.at[u], c_s[u])
            pbuf[u] = tcols_lo[u] * c_lo + tcols_hi[u] * c_hi
        lanes16 = lax.iota(jnp.int32, LANES)
        acc0 = jnp.zeros((LANES,), jnp.float32)
        acc1 = jnp.zeros((LANES,), jnp.float32)
        for j in range(0, LANES, 2):
            acc0 = acc0 + plsc.load_gather(
                pbuf, [lanes16, jnp.full((LANES,), j, jnp.int32)])
            acc1 = acc1 + plsc.load_gather(
                pbuf, [lanes16, jnp.full((LANES,), j + 1, jnp.int32)])
        out_v[pl.ds(g * LANES, LANES)] = acc0 + acc1
        return 0

    lax.fori_loop(0, B_PER_W // LANES, round16, 0)

    pltpu.sync_copy(out_v, out_hbm.at[pl.ds(wid * B_PER_W, B_PER_W)])


@jax.jit
def _run(t_idx, c_idx, t_tab, c_tab):
    mesh = plsc.VectorSubcoreMesh(core_axis_name="c", subcore_axis_name="s")
    return pl.kernel(
        _sc_body,
        out_type=jax.ShapeDtypeStruct((B,), jnp.float32),
        mesh=mesh,
        compiler_params=pltpu.CompilerParams(needs_layout_passes=False),
        scratch_types=[
            pltpu.VMEM((N_CHUNKS, IDX_CHUNK), jnp.int32),
            pltpu.VMEM((N_CHUNKS, IDX_CHUNK), jnp.int32),
            pltpu.VMEM((LANES, D, PANEL), jnp.float32),
            pltpu.VMEM((LANES, LANES), jnp.float32),
            pltpu.VMEM((LANES, LANES), jnp.float32),
            pltpu.VMEM((LANES, LANES), jnp.float32),
            pltpu.VMEM((D, TAIL_W), jnp.float32),
            pltpu.VMEM((D, TAIL_W), jnp.float32),
            pltpu.VMEM((B_PER_W,), jnp.float32),
            pltpu.SemaphoreType.DMA,
        ],
    )(t_idx, c_idx, t_tab, c_tab)


def kernel(t_kmer, c_kmer, label, T_weight, C_weight):
    del label
    t_idx = t_kmer.astype(jnp.int32).reshape(B // IDX_CHUNK, IDX_CHUNK)
    c_idx = c_kmer.astype(jnp.int32).reshape(B // IDX_CHUNK, IDX_CHUNK)
    return _run(t_idx, c_idx, T_weight.T, C_weight.T)


# speedup vs baseline: 3.5342x; 1.0562x over previous
import jax
import jax.numpy as jnp
from jax import lax
from jax.experimental import pallas as pl
from jax.experimental.pallas import tpu as pltpu
from jax.experimental.pallas import tpu_sc as plsc

B = 16384
D = 32
V = 1000000
LANES = 16
NUM_WORKERS = 32
B_PER_W = B // NUM_WORKERS
IDX_CHUNK = 128
N_CHUNKS = B_PER_W // IDX_CHUNK
PANEL = 128
TAIL_START = (V // PANEL) * PANEL
TAIL_W = V - TAIL_START
LAST_PANEL = TAIL_START - PANEL
CHUNK8 = 8


def _sc_body(t_idx_hbm, c_idx_hbm, t_tab_hbm, c_tab_hbm, out_hbm,
             t_idx_v, c_idx_v, buf_a, buf_b, pbuf,
             t_tail, c_tail, out_v, sem_a, sem_b):
    nc = 2
    wid = lax.axis_index("s") * nc + lax.axis_index("c")
    blk = wid * N_CHUNKS

    pltpu.sync_copy(t_idx_hbm.at[pl.ds(blk, N_CHUNKS)], t_idx_v)
    pltpu.sync_copy(c_idx_hbm.at[pl.ds(blk, N_CHUNKS)], c_idx_v)
    pltpu.sync_copy(t_tab_hbm.at[:, pl.ds(TAIL_START, TAIL_W)], t_tail)
    pltpu.sync_copy(c_tab_hbm.at[:, pl.ds(TAIL_START, TAIL_W)], c_tail)

    d_lo = lax.iota(jnp.int32, LANES)
    d_hi = d_lo + jnp.int32(LANES)

    def column(tail_ref, buf_slot, vs):
        col = jnp.minimum(vs & jnp.int32(PANEL - 1), jnp.int32(PANEL - 1))
        tcol = jnp.minimum(
            jnp.maximum(vs - jnp.int32(TAIL_START), jnp.int32(0)),
            jnp.int32(TAIL_W - 1))
        is_tail = jnp.broadcast_to(vs >= jnp.int32(TAIL_START), (LANES,))
        colv = jnp.broadcast_to(col, (LANES,))
        tcolv = jnp.broadcast_to(tcol, (LANES,))
        lo = jnp.where(
            is_tail,
            plsc.load_gather(tail_ref, [d_lo, tcolv]),
            plsc.load_gather(buf_slot, [d_lo, colv]))
        hi = jnp.where(
            is_tail,
            plsc.load_gather(tail_ref, [d_hi, tcolv]),
            plsc.load_gather(buf_slot, [d_hi, colv]))
        return lo, hi

    def panel_start(vs):
        return jnp.minimum(
            lax.shift_right_logical(vs, jnp.int32(7)) * jnp.int32(PANEL),
            jnp.int32(LAST_PANEL))

    H = LANES // 2

    def fire(tab_ref, scalars, buf, dsem):
        return [pltpu.async_copy(
            tab_ref.at[:, pl.ds(pl.multiple_of(panel_start(s), PANEL), PANEL)],
            buf.at[u], dsem) for u, s in enumerate(scalars)]

    def drain(buf, dsem):
        for u in range(H):
            pltpu.make_async_copy(
                t_tab_hbm.at[:, pl.ds(0, PANEL)], buf.at[u], dsem).wait()

    def scal(vec, lane0):
        return [lax.squeeze(lax.slice(vec, (lane0 + u,), (lane0 + u + 1,)),
                            (0,)) for u in range(H)]

    def halfround(h, tvec, cvec, tvec_n):
        t_s = scal(tvec, h * H)
        c_s = scal(cvec, h * H)
        t_next = scal(tvec if h == 0 else tvec_n, H - h * H)
        cps_c = fire(c_tab_hbm, c_s, buf_b, sem_b)
        drain(buf_a, sem_a)
        tc = [column(t_tail, buf_a.at[u], t_s[u]) for u in range(H)]
        fire(t_tab_hbm, t_next, buf_a, sem_a)
        for cp in cps_c:
            cp.wait()
        for u in range(H):
            c_lo, c_hi = column(c_tail, buf_b.at[u], c_s[u])
            pbuf[h * H + u] = tc[u][0] * c_lo + tc[u][1] * c_hi

    def round16(g, _):
        tvec = t_idx_v[g // 8, pl.ds((g % 8) * LANES, LANES)]
        cvec = c_idx_v[g // 8, pl.ds((g % 8) * LANES, LANES)]
        gn = jnp.minimum(g + 1, jnp.int32(B_PER_W // LANES - 1))
        tvec_n = t_idx_v[gn // 8, pl.ds((gn % 8) * LANES, LANES)]
        halfround(0, tvec, cvec, tvec_n)
        halfround(1, tvec, cvec, tvec_n)
        lanes16 = lax.iota(jnp.int32, LANES)
        acc0 = jnp.zeros((LANES,), jnp.float32)
        acc1 = jnp.zeros((LANES,), jnp.float32)
        for j in range(0, LANES, 2):
            acc0 = acc0 + plsc.load_gather(
                pbuf, [lanes16, jnp.full((LANES,), j, jnp.int32)])
            acc1 = acc1 + plsc.load_gather(
                pbuf, [lanes16, jnp.full((LANES,), j + 1, jnp.int32)])
        out_v[pl.ds(g * LANES, LANES)] = acc0 + acc1
        return 0

    tvec0 = t_idx_v[0, pl.ds(0, LANES)]
    fire(t_tab_hbm, scal(tvec0, 0), buf_a, sem_a)
    lax.fori_loop(0, B_PER_W // LANES, round16, 0)
    drain(buf_a, sem_a)

    pltpu.sync_copy(out_v, out_hbm.at[pl.ds(wid * B_PER_W, B_PER_W)])


@jax.jit
def _run(t_idx, c_idx, t_tab, c_tab):
    mesh = plsc.VectorSubcoreMesh(core_axis_name="c", subcore_axis_name="s")
    return pl.kernel(
        _sc_body,
        out_type=jax.ShapeDtypeStruct((B,), jnp.float32),
        mesh=mesh,
        compiler_params=pltpu.CompilerParams(needs_layout_passes=False),
        scratch_types=[
            pltpu.VMEM((N_CHUNKS, IDX_CHUNK), jnp.int32),
            pltpu.VMEM((N_CHUNKS, IDX_CHUNK), jnp.int32),
            pltpu.VMEM((LANES // 2, D, PANEL), jnp.float32),
            pltpu.VMEM((LANES // 2, D, PANEL), jnp.float32),
            pltpu.VMEM((LANES, LANES), jnp.float32),
            pltpu.VMEM((D, TAIL_W), jnp.float32),
            pltpu.VMEM((D, TAIL_W), jnp.float32),
            pltpu.VMEM((B_PER_W,), jnp.float32),
            pltpu.SemaphoreType.DMA,
            pltpu.SemaphoreType.DMA,
        ],
    )(t_idx, c_idx, t_tab, c_tab)


def kernel(t_kmer, c_kmer, label, T_weight, C_weight):
    del label
    t_idx = t_kmer.astype(jnp.int32).reshape(B // IDX_CHUNK, IDX_CHUNK)
    c_idx = c_kmer.astype(jnp.int32).reshape(B // IDX_CHUNK, IDX_CHUNK)
    return _run(t_idx, c_idx, T_weight.T, C_weight.T)
